# baseline (speedup 1.0000x reference)
import functools

import jax
import jax.numpy as jnp
from jax.experimental import pallas as pl
from jax.experimental.pallas import tpu as pltpu

_TOPK = 13
_EPS = 1e-9
_IOU_EPS = 1e-10
_NPAD = 56
_ZWIN = 128


def _assign_body(n_real, L,
                 gtb_ref, gtbt_ref, gpt_ref, glab_ref, pk_ref, bg_ref,
                 scal_ref, bb_ref, pose_ref):
    g = gtb_ref[0]
    pk = pk_ref[0]

    x1g = g[:, 0:1]
    y1g = g[:, 1:2]
    x2g = g[:, 2:3]
    y2g = g[:, 3:4]
    px1 = pk[0:1, :]
    py1 = pk[1:2, :]
    px2 = pk[2:3, :]
    py2 = pk[3:4, :]
    ax = pk[4:5, :]
    ay = pk[5:6, :]
    s = pk[6:7, :]

    ix1 = jnp.maximum(x1g, px1)
    iy1 = jnp.maximum(y1g, py1)
    ix2 = jnp.minimum(x2g, px2)
    iy2 = jnp.minimum(y2g, py2)
    overlap = jnp.maximum(ix2 - ix1, 0.0) * jnp.maximum(iy2 - iy1, 0.0)
    area1 = jnp.maximum(x2g - x1g, 0.0) * jnp.maximum(y2g - y1g, 0.0)
    area2 = jnp.maximum(px2 - px1, 0.0) * jnp.maximum(py2 - py1, 0.0)
    iou = overlap / (area1 + area2 - overlap + _IOU_EPS)

    dmin = jnp.minimum(jnp.minimum(ax - x1g, ay - y1g),
                       jnp.minimum(x2g - ax, y2g - ay))
    in_gts = dmin > _EPS

    iou2 = iou * iou
    am = s * (iou2 * iou2 * iou2)
    mt = jnp.where(in_gts, am, 0.0)

    work = mt
    for _ in range(_TOPK - 1):
        mx = jnp.max(work, axis=1, keepdims=True)
        work = jnp.where(work == mx, -1.0, work)
    thr = jnp.max(work, axis=1, keepdims=True)
    marked = jnp.logical_and(mt >= thr, mt > 0.0)
    npos = jnp.sum(marked.astype(jnp.float32), axis=1, keepdims=True)
    needed = _TOPK - npos

    zmask = (mt[:, :_ZWIN] == 0.0).astype(jnp.float32)
    tri = (jax.lax.broadcasted_iota(jnp.int32, (_ZWIN, _ZWIN), 0)
           <= jax.lax.broadcasted_iota(jnp.int32, (_ZWIN, _ZWIN), 1))
    zrank = jax.lax.dot_general(zmask, tri.astype(jnp.float32),
                                (((1,), (0,)), ((), ())),
                                preferred_element_type=jnp.float32)
    sel_zero = jnp.logical_and(zmask > 0.0, zrank <= needed)
    topk = jnp.concatenate(
        [jnp.logical_or(marked[:, :_ZWIN], sel_zero), marked[:, _ZWIN:]],
        axis=1)

    mask_pre = jnp.logical_and(topk, in_gts)
    sum_pre = jnp.sum(mask_pre.astype(jnp.float32), axis=0, keepdims=True)
    multi = sum_pre > 1.0

    sub = jax.lax.broadcasted_iota(jnp.int32, (_NPAD, L), 0)
    mxi = jnp.max(iou, axis=0, keepdims=True)
    firsti = jnp.min(jnp.where(iou == mxi, sub, _NPAD), axis=0, keepdims=True)
    onehot_iou = sub == firsti

    mask = jnp.logical_or(jnp.logical_and(multi, onehot_iou),
                          jnp.logical_and(jnp.logical_not(multi), mask_pre))
    aidx0 = jnp.min(jnp.where(mask, sub, _NPAD), axis=0, keepdims=True)
    pos = aidx0 < _NPAD
    aidx = jnp.where(pos, aidx0, 0)

    amm = jnp.where(mask, am, 0.0)
    ioum = jnp.where(mask, iou, 0.0)
    mm = jnp.max(amm, axis=1, keepdims=True)
    mi = jnp.max(ioum, axis=1, keepdims=True)
    scale = mi / (mm + _EPS)
    t = jnp.max(amm * scale, axis=0, keepdims=True)

    onehot = (sub == aidx).astype(jnp.float32)

    glab = glab_ref[0]
    labf = jax.lax.dot_general(glab, onehot, (((1,), (0,)), ((), ())),
                               preferred_element_type=jnp.float32)
    bg = bg_ref[0, 0].astype(jnp.float32)
    lab = jnp.where(pos, labf, bg)
    sc = jnp.where(lab == 0.0, t, 0.0)
    b = pl.program_id(0)
    idxf = (aidx + b * n_real).astype(jnp.float32)
    zero_rows = jnp.zeros((5, L), jnp.float32)
    scal_ref[0] = jnp.concatenate([lab, sc, idxf, zero_rows], axis=0)

    gbt = gtbt_ref[0]
    bb_ref[0] = jax.lax.dot_general(gbt, onehot, (((1,), (0,)), ((), ())),
                                    preferred_element_type=jnp.float32)
    gpt = gpt_ref[0]
    pose_ref[0] = jax.lax.dot_general(gpt, onehot, (((1,), (0,)), ((), ())),
                                      preferred_element_type=jnp.float32)


def kernel(pred_scores, pred_bboxes, pred_poses, anchor_points,
           num_anchors_list, gt_labels, gt_bboxes, gt_poses, pad_gt_mask,
           bg_index):
    B, L, C = pred_scores.shape
    n = gt_bboxes.shape[1]
    K = gt_poses.shape[2]
    K3 = K * 3

    gtb = jnp.pad(gt_bboxes, ((0, 0), (0, _NPAD - n), (0, 0)))
    gtbt = jnp.transpose(gtb, (0, 2, 1))
    gpt = jnp.transpose(
        jnp.pad(gt_poses.reshape(B, n, K3), ((0, 0), (0, _NPAD - n), (0, 0))),
        (0, 2, 1))
    glab = jnp.pad(gt_labels.astype(jnp.float32), ((0, 0), (0, _NPAD - n), (0, 0)))
    glab = jnp.reshape(glab, (B, 1, _NPAD))
    pbbt = jnp.transpose(pred_bboxes, (0, 2, 1))
    anch = jnp.broadcast_to(anchor_points.T[None], (B, 2, L))
    ps = jnp.reshape(pred_scores[:, :, 0], (B, 1, L))
    pk = jnp.concatenate(
        [pbbt, anch, ps, jnp.zeros((B, 1, L), jnp.float32)], axis=1)
    bg = jnp.reshape(jnp.asarray(bg_index, jnp.int32), (1, 1))

    body = functools.partial(_assign_body, n, L)
    scal, bb, pose = pl.pallas_call(
        body,
        grid=(B,),
        in_specs=[
            pl.BlockSpec((1, _NPAD, 4), lambda b: (b, 0, 0)),
            pl.BlockSpec((1, 4, _NPAD), lambda b: (b, 0, 0)),
            pl.BlockSpec((1, K3, _NPAD), lambda b: (b, 0, 0)),
            pl.BlockSpec((1, 1, _NPAD), lambda b: (b, 0, 0)),
            pl.BlockSpec((1, 8, L), lambda b: (b, 0, 0)),
            pl.BlockSpec(memory_space=pltpu.SMEM),
        ],
        out_specs=[
            pl.BlockSpec((1, 8, L), lambda b: (b, 0, 0)),
            pl.BlockSpec((1, 4, L), lambda b: (b, 0, 0)),
            pl.BlockSpec((1, K3, L), lambda b: (b, 0, 0)),
        ],
        out_shape=[
            jax.ShapeDtypeStruct((B, 8, L), jnp.float32),
            jax.ShapeDtypeStruct((B, 4, L), jnp.float32),
            jax.ShapeDtypeStruct((B, K3, L), jnp.float32),
        ],
    )(gtb, gtbt, gpt, glab, pk, bg)

    assigned_labels = scal[:, 0, :].astype(jnp.int32)
    assigned_scores = jnp.reshape(scal[:, 1, :], (B, L, 1))
    assigned_gt_index = scal[:, 2, :].astype(jnp.int32)
    assigned_poses = jnp.reshape(jnp.transpose(pose, (0, 2, 1)), (B, L, K, 3))
    assigned_bboxes = jnp.transpose(bb, (0, 2, 1))
    return (assigned_labels, assigned_bboxes, assigned_poses, assigned_scores,
            assigned_gt_index)


# speedup vs baseline: 1.0105x; 1.0105x over previous
import functools

import jax
import jax.numpy as jnp
from jax.experimental import pallas as pl
from jax.experimental.pallas import tpu as pltpu

_TOPK = 13
_EPS = 1e-9
_IOU_EPS = 1e-10
_NPAD = 56
_ZWIN = 128


def _assign_body(n_real, L,
                 gtb_ref, gtbt_ref, gpt_ref, glab_ref, pk_ref, bg_ref,
                 lab_ref, sc_ref, idx_ref, bb_ref, pose_ref):
    g = gtb_ref[0]
    pk = pk_ref[0]

    x1g = g[:, 0:1]
    y1g = g[:, 1:2]
    x2g = g[:, 2:3]
    y2g = g[:, 3:4]
    px1 = pk[0:1, :]
    py1 = pk[1:2, :]
    px2 = pk[2:3, :]
    py2 = pk[3:4, :]
    ax = pk[4:5, :]
    ay = pk[5:6, :]
    s = pk[6:7, :]

    ix1 = jnp.maximum(x1g, px1)
    iy1 = jnp.maximum(y1g, py1)
    ix2 = jnp.minimum(x2g, px2)
    iy2 = jnp.minimum(y2g, py2)
    overlap = jnp.maximum(ix2 - ix1, 0.0) * jnp.maximum(iy2 - iy1, 0.0)
    area1 = jnp.maximum(x2g - x1g, 0.0) * jnp.maximum(y2g - y1g, 0.0)
    area2 = jnp.maximum(px2 - px1, 0.0) * jnp.maximum(py2 - py1, 0.0)
    iou = overlap / (area1 + area2 - overlap + _IOU_EPS)

    dmin = jnp.minimum(jnp.minimum(ax - x1g, ay - y1g),
                       jnp.minimum(x2g - ax, y2g - ay))
    in_gts = dmin > _EPS

    iou2 = iou * iou
    am = s * (iou2 * iou2 * iou2)
    mt = jnp.where(in_gts, am, 0.0)

    work = mt
    for _ in range(_TOPK - 1):
        mx = jnp.max(work, axis=1, keepdims=True)
        work = jnp.where(work == mx, -1.0, work)
    thr = jnp.max(work, axis=1, keepdims=True)
    marked = jnp.logical_and(mt >= thr, mt > 0.0)
    npos = jnp.sum(marked.astype(jnp.float32), axis=1, keepdims=True)
    needed = _TOPK - npos

    zmask = (mt[:, :_ZWIN] == 0.0).astype(jnp.float32)
    tri = (jax.lax.broadcasted_iota(jnp.int32, (_ZWIN, _ZWIN), 0)
           <= jax.lax.broadcasted_iota(jnp.int32, (_ZWIN, _ZWIN), 1))
    zrank = jax.lax.dot_general(zmask, tri.astype(jnp.float32),
                                (((1,), (0,)), ((), ())),
                                preferred_element_type=jnp.float32)
    sel_zero = jnp.logical_and(zmask > 0.0, zrank <= needed)
    topk = jnp.concatenate(
        [jnp.logical_or(marked[:, :_ZWIN], sel_zero), marked[:, _ZWIN:]],
        axis=1)

    mask_pre = jnp.logical_and(topk, in_gts)
    sum_pre = jnp.sum(mask_pre.astype(jnp.float32), axis=0, keepdims=True)
    multi = sum_pre > 1.0

    sub = jax.lax.broadcasted_iota(jnp.int32, (_NPAD, L), 0)
    mxi = jnp.max(iou, axis=0, keepdims=True)
    firsti = jnp.min(jnp.where(iou == mxi, sub, _NPAD), axis=0, keepdims=True)
    onehot_iou = sub == firsti

    mask = jnp.logical_or(jnp.logical_and(multi, onehot_iou),
                          jnp.logical_and(jnp.logical_not(multi), mask_pre))
    aidx0 = jnp.min(jnp.where(mask, sub, _NPAD), axis=0, keepdims=True)
    pos = aidx0 < _NPAD
    aidx = jnp.where(pos, aidx0, 0)

    amm = jnp.where(mask, am, 0.0)
    ioum = jnp.where(mask, iou, 0.0)
    mm = jnp.max(amm, axis=1, keepdims=True)
    mi = jnp.max(ioum, axis=1, keepdims=True)
    scale = mi / (mm + _EPS)
    t = jnp.max(amm * scale, axis=0, keepdims=True)

    onehot = (sub == aidx).astype(jnp.float32)

    glab = glab_ref[0]
    labf = jax.lax.dot_general(glab, onehot, (((1,), (0,)), ((), ())),
                               preferred_element_type=jnp.float32)
    lab_g = labf.astype(jnp.int32)
    bg = bg_ref[0, 0]
    lab = jnp.where(pos, lab_g, bg)
    lab_ref[0] = lab
    sc_ref[0] = jnp.where(lab == 0, t, 0.0)
    b = pl.program_id(0)
    idx_ref[0] = aidx + b * n_real

    gbt = gtbt_ref[0]
    bb_ref[0] = jax.lax.dot_general(gbt, onehot, (((1,), (0,)), ((), ())),
                                    preferred_element_type=jnp.float32)
    gpt = gpt_ref[0]
    pose_ref[0] = jax.lax.dot_general(gpt, onehot, (((1,), (0,)), ((), ())),
                                      preferred_element_type=jnp.float32)


def kernel(pred_scores, pred_bboxes, pred_poses, anchor_points,
           num_anchors_list, gt_labels, gt_bboxes, gt_poses, pad_gt_mask,
           bg_index):
    B, L, C = pred_scores.shape
    n = gt_bboxes.shape[1]
    K = gt_poses.shape[2]
    K3 = K * 3

    gtb = jnp.pad(gt_bboxes, ((0, 0), (0, _NPAD - n), (0, 0)))
    gtbt = jnp.transpose(gtb, (0, 2, 1))
    gpt = jnp.transpose(
        jnp.pad(gt_poses.reshape(B, n, K3), ((0, 0), (0, _NPAD - n), (0, 0))),
        (0, 2, 1))
    glab = jnp.pad(gt_labels.astype(jnp.float32), ((0, 0), (0, _NPAD - n), (0, 0)))
    glab = jnp.reshape(glab, (B, 1, _NPAD))
    pbbt = jnp.transpose(pred_bboxes, (0, 2, 1))
    anch = jnp.broadcast_to(anchor_points.T[None], (B, 2, L))
    ps = jnp.reshape(pred_scores[:, :, 0], (B, 1, L))
    pk = jnp.concatenate(
        [pbbt, anch, ps, jnp.zeros((B, 1, L), jnp.float32)], axis=1)
    bg = jnp.reshape(jnp.asarray(bg_index, jnp.int32), (1, 1))

    body = functools.partial(_assign_body, n, L)
    lab_r, sc_r, idx_r, bb, pose = pl.pallas_call(
        body,
        grid=(B,),
        in_specs=[
            pl.BlockSpec((1, _NPAD, 4), lambda b: (b, 0, 0)),
            pl.BlockSpec((1, 4, _NPAD), lambda b: (b, 0, 0)),
            pl.BlockSpec((1, K3, _NPAD), lambda b: (b, 0, 0)),
            pl.BlockSpec((1, 1, _NPAD), lambda b: (b, 0, 0)),
            pl.BlockSpec((1, 8, L), lambda b: (b, 0, 0)),
            pl.BlockSpec(memory_space=pltpu.SMEM),
        ],
        out_specs=[
            pl.BlockSpec((1, 1, L), lambda b: (b, 0, 0)),
            pl.BlockSpec((1, 1, L), lambda b: (b, 0, 0)),
            pl.BlockSpec((1, 1, L), lambda b: (b, 0, 0)),
            pl.BlockSpec((1, 4, L), lambda b: (b, 0, 0)),
            pl.BlockSpec((1, K3, L), lambda b: (b, 0, 0)),
        ],
        out_shape=[
            jax.ShapeDtypeStruct((B, 1, L), jnp.int32),
            jax.ShapeDtypeStruct((B, 1, L), jnp.float32),
            jax.ShapeDtypeStruct((B, 1, L), jnp.int32),
            jax.ShapeDtypeStruct((B, 4, L), jnp.float32),
            jax.ShapeDtypeStruct((B, K3, L), jnp.float32),
        ],
    )(gtb, gtbt, gpt, glab, pk, bg)

    assigned_labels = jnp.reshape(lab_r, (B, L))
    assigned_scores = jnp.reshape(sc_r, (B, L, 1))
    assigned_gt_index = jnp.reshape(idx_r, (B, L))
    assigned_poses = jnp.reshape(jnp.transpose(pose, (0, 2, 1)), (B, L, K, 3))
    assigned_bboxes = jnp.transpose(bb, (0, 2, 1))
    return (assigned_labels, assigned_bboxes, assigned_poses, assigned_scores,
            assigned_gt_index)


# speedup vs baseline: 1.0374x; 1.0267x over previous
import functools

import jax
import jax.numpy as jnp
from jax.experimental import pallas as pl
from jax.experimental.pallas import tpu as pltpu

_TOPK = 13
_EPS = 1e-9
_IOU_EPS = 1e-10
_NPAD = 56
_ZWIN = 128


def _assign_body(n_real, L,
                 gtb_ref, gtbt_ref, gpt_ref, glab_ref, pbbt_ref, anch_ref,
                 ps_ref, bg_ref,
                 lab_ref, sc_ref, idx_ref, bb_ref, pose_ref):
    g = gtb_ref[0]
    p = pbbt_ref[0]
    a = anch_ref[...]
    s = ps_ref[0]

    x1g = g[:, 0:1]
    y1g = g[:, 1:2]
    x2g = g[:, 2:3]
    y2g = g[:, 3:4]
    px1 = p[0:1, :]
    py1 = p[1:2, :]
    px2 = p[2:3, :]
    py2 = p[3:4, :]
    ax = a[0:1, :]
    ay = a[1:2, :]

    ix1 = jnp.maximum(x1g, px1)
    iy1 = jnp.maximum(y1g, py1)
    ix2 = jnp.minimum(x2g, px2)
    iy2 = jnp.minimum(y2g, py2)
    overlap = jnp.maximum(ix2 - ix1, 0.0) * jnp.maximum(iy2 - iy1, 0.0)
    area1 = jnp.maximum(x2g - x1g, 0.0) * jnp.maximum(y2g - y1g, 0.0)
    area2 = jnp.maximum(px2 - px1, 0.0) * jnp.maximum(py2 - py1, 0.0)
    iou = overlap / (area1 + area2 - overlap + _IOU_EPS)

    dmin = jnp.minimum(jnp.minimum(ax - x1g, ay - y1g),
                       jnp.minimum(x2g - ax, y2g - ay))
    in_gts = dmin > _EPS

    iou2 = iou * iou
    am = s * (iou2 * iou2 * iou2)
    mt = jnp.where(in_gts, am, 0.0)

    work = mt
    for _ in range(_TOPK - 1):
        mx = jnp.max(work, axis=1, keepdims=True)
        work = jnp.where(work == mx, -1.0, work)
    thr = jnp.max(work, axis=1, keepdims=True)
    marked = jnp.logical_and(mt >= thr, mt > 0.0)
    npos = jnp.sum(marked.astype(jnp.float32), axis=1, keepdims=True)
    needed = _TOPK - npos

    zmask = (mt[:, :_ZWIN] == 0.0).astype(jnp.float32)
    tri = (jax.lax.broadcasted_iota(jnp.int32, (_ZWIN, _ZWIN), 0)
           <= jax.lax.broadcasted_iota(jnp.int32, (_ZWIN, _ZWIN), 1))
    zrank = jax.lax.dot_general(zmask, tri.astype(jnp.float32),
                                (((1,), (0,)), ((), ())),
                                preferred_element_type=jnp.float32)
    sel_zero = jnp.logical_and(zmask > 0.0, zrank <= needed)
    topk = jnp.concatenate(
        [jnp.logical_or(marked[:, :_ZWIN], sel_zero), marked[:, _ZWIN:]],
        axis=1)

    mask_pre = jnp.logical_and(topk, in_gts)
    sum_pre = jnp.sum(mask_pre.astype(jnp.float32), axis=0, keepdims=True)
    multi = sum_pre > 1.0

    sub = jax.lax.broadcasted_iota(jnp.int32, (_NPAD, L), 0)
    mxi = jnp.max(iou, axis=0, keepdims=True)
    firsti = jnp.min(jnp.where(iou == mxi, sub, _NPAD), axis=0, keepdims=True)
    onehot_iou = sub == firsti

    mask = jnp.logical_or(jnp.logical_and(multi, onehot_iou),
                          jnp.logical_and(jnp.logical_not(multi), mask_pre))
    aidx0 = jnp.min(jnp.where(mask, sub, _NPAD), axis=0, keepdims=True)
    pos = aidx0 < _NPAD
    aidx = jnp.where(pos, aidx0, 0)

    amm = jnp.where(mask, am, 0.0)
    ioum = jnp.where(mask, iou, 0.0)
    mm = jnp.max(amm, axis=1, keepdims=True)
    mi = jnp.max(ioum, axis=1, keepdims=True)
    scale = mi / (mm + _EPS)
    t = jnp.max(amm * scale, axis=0, keepdims=True)

    onehot = (sub == aidx).astype(jnp.float32)

    glab = glab_ref[0]
    labf = jax.lax.dot_general(glab, onehot, (((1,), (0,)), ((), ())),
                               preferred_element_type=jnp.float32)
    lab_g = labf.astype(jnp.int32)
    bg = bg_ref[0, 0]
    lab = jnp.where(pos, lab_g, bg)
    lab_ref[0] = lab
    sc_ref[0] = jnp.where(lab == 0, t, 0.0)
    b = pl.program_id(0)
    idx_ref[0] = aidx + b * n_real

    gbt = gtbt_ref[0]
    bb_ref[0] = jax.lax.dot_general(gbt, onehot, (((1,), (0,)), ((), ())),
                                    preferred_element_type=jnp.float32)
    gpt = gpt_ref[0]
    pose_ref[0] = jax.lax.dot_general(gpt, onehot, (((1,), (0,)), ((), ())),
                                      preferred_element_type=jnp.float32)


def kernel(pred_scores, pred_bboxes, pred_poses, anchor_points,
           num_anchors_list, gt_labels, gt_bboxes, gt_poses, pad_gt_mask,
           bg_index):
    B, L, C = pred_scores.shape
    n = gt_bboxes.shape[1]
    K = gt_poses.shape[2]
    K3 = K * 3

    gtb = jnp.pad(gt_bboxes, ((0, 0), (0, _NPAD - n), (0, 0)))
    gtbt = jnp.transpose(gtb, (0, 2, 1))
    gpt = jnp.transpose(
        jnp.pad(gt_poses.reshape(B, n, K3), ((0, 0), (0, _NPAD - n), (0, 0))),
        (0, 2, 1))
    glab = jnp.pad(gt_labels.astype(jnp.float32), ((0, 0), (0, _NPAD - n), (0, 0)))
    glab = jnp.reshape(glab, (B, 1, _NPAD))
    pbbt = jnp.transpose(pred_bboxes, (0, 2, 1))
    anch = anchor_points.T
    ps = jnp.reshape(pred_scores[:, :, 0], (B, 1, L))
    bg = jnp.reshape(jnp.asarray(bg_index, jnp.int32), (1, 1))

    body = functools.partial(_assign_body, n, L)
    lab_r, sc_r, idx_r, bb, pose = pl.pallas_call(
        body,
        grid=(B,),
        in_specs=[
            pl.BlockSpec((1, _NPAD, 4), lambda b: (b, 0, 0)),
            pl.BlockSpec((1, 4, _NPAD), lambda b: (b, 0, 0)),
            pl.BlockSpec((1, K3, _NPAD), lambda b: (b, 0, 0)),
            pl.BlockSpec((1, 1, _NPAD), lambda b: (b, 0, 0)),
            pl.BlockSpec((1, 4, L), lambda b: (b, 0, 0)),
            pl.BlockSpec((2, L), lambda b: (0, 0)),
            pl.BlockSpec((1, 1, L), lambda b: (b, 0, 0)),
            pl.BlockSpec(memory_space=pltpu.SMEM),
        ],
        out_specs=[
            pl.BlockSpec((1, 1, L), lambda b: (b, 0, 0)),
            pl.BlockSpec((1, 1, L), lambda b: (b, 0, 0)),
            pl.BlockSpec((1, 1, L), lambda b: (b, 0, 0)),
            pl.BlockSpec((1, 4, L), lambda b: (b, 0, 0)),
            pl.BlockSpec((1, K3, L), lambda b: (b, 0, 0)),
        ],
        out_shape=[
            jax.ShapeDtypeStruct((B, 1, L), jnp.int32),
            jax.ShapeDtypeStruct((B, 1, L), jnp.float32),
            jax.ShapeDtypeStruct((B, 1, L), jnp.int32),
            jax.ShapeDtypeStruct((B, 4, L), jnp.float32),
            jax.ShapeDtypeStruct((B, K3, L), jnp.float32),
        ],
    )(gtb, gtbt, gpt, glab, pbbt, anch, ps, bg)

    assigned_labels = jnp.reshape(lab_r, (B, L))
    assigned_scores = jnp.reshape(sc_r, (B, L, 1))
    assigned_gt_index = jnp.reshape(idx_r, (B, L))
    assigned_poses = jnp.reshape(jnp.transpose(pose, (0, 2, 1)), (B, L, K, 3))
    assigned_bboxes = jnp.transpose(bb, (0, 2, 1))
    return (assigned_labels, assigned_bboxes, assigned_poses, assigned_scores,
            assigned_gt_index)
